# baseline (speedup 1.0000x reference)
import jax
import jax.numpy as jnp
from jax.experimental import pallas as pl

_B, _S, _D, _E, _K = 4, 4096, 4096, 64, 8
_T = 256


def _router_kernel(x_ref, w_ref, b_ref, logits_ref, weights_ref, experts_ref):
    x = x_ref[...]
    w = w_ref[...]
    logits = jnp.dot(x, w, preferred_element_type=jnp.float32)
    logits = logits + b_ref[...]
    logits_ref[...] = logits

    bits = jax.lax.bitcast_convert_type(logits, jnp.int32)
    key = jnp.where(bits < 0, bits ^ jnp.int32(0x7FFFFFFF), bits)
    lane = jax.lax.broadcasted_iota(jnp.int32, logits.shape, 1)
    packed = (key & jnp.int32(~0x3F)) | (jnp.int32(_E - 1) - lane)

    kiota = jax.lax.broadcasted_iota(jnp.int32, (logits.shape[0], _K), 1)
    top_vals = jnp.zeros((logits.shape[0], _K), jnp.float32)
    top_idx = jnp.zeros((logits.shape[0], _K), jnp.int32)
    cur = packed
    imin = jnp.int32(-0x80000000)
    for k in range(_K):
        m = jnp.max(cur, axis=1, keepdims=True)
        cur = jnp.where(cur == m, imin, cur)
        idx_k = jnp.int32(_E - 1) - (m & jnp.int32(0x3F))
        keybits = m | jnp.int32(0x3F)
        vbits = jnp.where(keybits < 0, keybits ^ jnp.int32(0x7FFFFFFF), keybits)
        val_k = jax.lax.bitcast_convert_type(vbits, jnp.float32)
        top_idx = jnp.where(kiota == k, idx_k, top_idx)
        top_vals = jnp.where(kiota == k, val_k, top_vals)

    e = jnp.exp(top_vals - top_vals[:, 0:1])
    weights_ref[...] = e / jnp.sum(e, axis=1, keepdims=True)
    experts_ref[...] = top_idx


def kernel(hidden_states, gate_w, gate_b):
    B, S, D = hidden_states.shape
    E = gate_w.shape[0]
    N = B * S
    x = hidden_states.reshape(N, D)
    wt = gate_w.T
    b = gate_b.reshape(1, E)

    grid = (N // _T,)
    logits, weights, experts = pl.pallas_call(
        _router_kernel,
        grid=grid,
        in_specs=[
            pl.BlockSpec((_T, D), lambda i: (i, 0)),
            pl.BlockSpec((D, E), lambda i: (0, 0)),
            pl.BlockSpec((1, E), lambda i: (0, 0)),
        ],
        out_specs=[
            pl.BlockSpec((_T, E), lambda i: (i, 0)),
            pl.BlockSpec((_T, _K), lambda i: (i, 0)),
            pl.BlockSpec((_T, _K), lambda i: (i, 0)),
        ],
        out_shape=[
            jax.ShapeDtypeStruct((N, E), jnp.float32),
            jax.ShapeDtypeStruct((N, _K), jnp.float32),
            jax.ShapeDtypeStruct((N, _K), jnp.int32),
        ],
    )(x, wt, b)

    return (weights.reshape(B, S, _K),
            experts.reshape(B, S, _K),
            logits.reshape(B, S, E))


# speedup vs baseline: 1.4881x; 1.4881x over previous
import jax
import jax.numpy as jnp
from jax.experimental import pallas as pl

_B, _S, _D, _E, _K = 4, 4096, 4096, 64, 8
_T = 256


def _router_kernel(x_ref, w_ref, b_ref, logits_ref, weights_ref,
                   experts_ref):
    x = x_ref[...]
    w = w_ref[...]
    logits = jnp.dot(x, w, preferred_element_type=jnp.float32)
    logits = logits + b_ref[...]
    logits_ref[...] = logits

    lt = logits.T
    bits = jax.lax.bitcast_convert_type(lt, jnp.int32)
    key = jnp.where(bits < 0, bits ^ jnp.int32(0x7FFFFFFF), bits)
    row = jax.lax.broadcasted_iota(jnp.int32, lt.shape, 0)
    packed = (key & jnp.int32(~0x3F)) | (jnp.int32(_E - 1) - row)

    kiota = jax.lax.broadcasted_iota(jnp.int32, (_K, lt.shape[1]), 0)
    top_vals = jnp.zeros((_K, lt.shape[1]), jnp.float32)
    top_idx = jnp.zeros((_K, lt.shape[1]), jnp.int32)
    cur = packed
    imin = jnp.int32(-0x80000000)
    for k in range(_K):
        m = jnp.max(cur, axis=0, keepdims=True)
        cur = jnp.where(cur == m, imin, cur)
        idx_k = jnp.int32(_E - 1) - (m & jnp.int32(0x3F))
        keybits = m | jnp.int32(0x3F)
        vbits = jnp.where(keybits < 0, keybits ^ jnp.int32(0x7FFFFFFF), keybits)
        val_k = jax.lax.bitcast_convert_type(vbits, jnp.float32)
        top_idx = jnp.where(kiota == k, idx_k, top_idx)
        top_vals = jnp.where(kiota == k, val_k, top_vals)

    e = jnp.exp(top_vals - top_vals[0:1, :])
    weights_ref[...] = e / jnp.sum(e, axis=0, keepdims=True)
    experts_ref[...] = top_idx


def kernel(hidden_states, gate_w, gate_b):
    B, S, D = hidden_states.shape
    E = gate_w.shape[0]
    N = B * S
    x = hidden_states.reshape(N, D)
    wt = gate_w.T
    b = gate_b.reshape(1, E)

    grid = (N // _T,)
    logits, weights_t, experts_t = pl.pallas_call(
        _router_kernel,
        grid=grid,
        in_specs=[
            pl.BlockSpec((_T, D), lambda i: (i, 0)),
            pl.BlockSpec((D, E), lambda i: (0, 0)),
            pl.BlockSpec((1, E), lambda i: (0, 0)),
        ],
        out_specs=[
            pl.BlockSpec((_T, E), lambda i: (i, 0)),
            pl.BlockSpec((_K, _T), lambda i: (0, i)),
            pl.BlockSpec((_K, _T), lambda i: (0, i)),
        ],
        out_shape=[
            jax.ShapeDtypeStruct((N, E), jnp.float32),
            jax.ShapeDtypeStruct((_K, N), jnp.float32),
            jax.ShapeDtypeStruct((_K, N), jnp.int32),
        ],
    )(x, wt, b)

    return (weights_t.T.reshape(B, S, _K),
            experts_t.T.reshape(B, S, _K),
            logits.reshape(B, S, E))


# speedup vs baseline: 1.7659x; 1.1867x over previous
import jax
import jax.numpy as jnp
from jax.experimental import pallas as pl

_B, _S, _D, _E, _K = 4, 4096, 4096, 64, 8
_T = 512


def _router_kernel(x_ref, w_ref, b_ref, logits_ref, weights_ref,
                   experts_ref):
    x = x_ref[...]
    w = w_ref[...]
    logits = jnp.dot(x, w, preferred_element_type=jnp.float32)
    logits = logits + b_ref[...]
    logits_ref[...] = logits

    lt = logits.T
    bits = jax.lax.bitcast_convert_type(lt, jnp.int32)
    key = jnp.where(bits < 0, bits ^ jnp.int32(0x7FFFFFFF), bits)
    row = jax.lax.broadcasted_iota(jnp.int32, lt.shape, 0)
    packed = (key & jnp.int32(~0x3F)) | (jnp.int32(_E - 1) - row)

    kiota = jax.lax.broadcasted_iota(jnp.int32, (_K, lt.shape[1]), 0)
    top_vals = jnp.zeros((_K, lt.shape[1]), jnp.float32)
    top_idx = jnp.zeros((_K, lt.shape[1]), jnp.int32)
    cur = packed
    imin = jnp.int32(-0x80000000)
    for k in range(_K):
        m = jnp.max(cur, axis=0, keepdims=True)
        cur = jnp.where(cur == m, imin, cur)
        idx_k = jnp.int32(_E - 1) - (m & jnp.int32(0x3F))
        keybits = m | jnp.int32(0x3F)
        vbits = jnp.where(keybits < 0, keybits ^ jnp.int32(0x7FFFFFFF), keybits)
        val_k = jax.lax.bitcast_convert_type(vbits, jnp.float32)
        top_idx = jnp.where(kiota == k, idx_k, top_idx)
        top_vals = jnp.where(kiota == k, val_k, top_vals)

    e = jnp.exp(top_vals - top_vals[0:1, :])
    weights_ref[...] = e / jnp.sum(e, axis=0, keepdims=True)
    experts_ref[...] = top_idx


def kernel(hidden_states, gate_w, gate_b):
    B, S, D = hidden_states.shape
    E = gate_w.shape[0]
    N = B * S
    x = hidden_states.reshape(N, D)
    wt = gate_w.T
    b = gate_b.reshape(1, E)

    grid = (N // _T,)
    logits, weights_t, experts_t = pl.pallas_call(
        _router_kernel,
        grid=grid,
        in_specs=[
            pl.BlockSpec((_T, D), lambda i: (i, 0)),
            pl.BlockSpec((D, E), lambda i: (0, 0)),
            pl.BlockSpec((1, E), lambda i: (0, 0)),
        ],
        out_specs=[
            pl.BlockSpec((_T, E), lambda i: (i, 0)),
            pl.BlockSpec((_K, _T), lambda i: (0, i)),
            pl.BlockSpec((_K, _T), lambda i: (0, i)),
        ],
        out_shape=[
            jax.ShapeDtypeStruct((N, E), jnp.float32),
            jax.ShapeDtypeStruct((_K, N), jnp.float32),
            jax.ShapeDtypeStruct((_K, N), jnp.int32),
        ],
    )(x, wt, b)

    return (weights_t.T.reshape(B, S, _K),
            experts_t.T.reshape(B, S, _K),
            logits.reshape(B, S, E))


# speedup vs baseline: 1.7960x; 1.0170x over previous
import jax
import jax.numpy as jnp
from jax.experimental import pallas as pl

_B, _S, _D, _E, _K = 4, 4096, 4096, 64, 8
_T = 1024


def _router_kernel(x_ref, w_ref, b_ref, logits_ref, weights_ref,
                   experts_ref):
    x = x_ref[...]
    w = w_ref[...]
    logits = jnp.dot(x, w, preferred_element_type=jnp.float32)
    logits = logits + b_ref[...]
    logits_ref[...] = logits

    lt = logits.T
    bits = jax.lax.bitcast_convert_type(lt, jnp.int32)
    key = jnp.where(bits < 0, bits ^ jnp.int32(0x7FFFFFFF), bits)
    row = jax.lax.broadcasted_iota(jnp.int32, lt.shape, 0)
    packed = (key & jnp.int32(~0x3F)) | (jnp.int32(_E - 1) - row)

    kiota = jax.lax.broadcasted_iota(jnp.int32, (_K, lt.shape[1]), 0)
    top_vals = jnp.zeros((_K, lt.shape[1]), jnp.float32)
    top_idx = jnp.zeros((_K, lt.shape[1]), jnp.int32)
    cur = packed
    imin = jnp.int32(-0x80000000)
    for k in range(_K):
        m = jnp.max(cur, axis=0, keepdims=True)
        cur = jnp.where(cur == m, imin, cur)
        idx_k = jnp.int32(_E - 1) - (m & jnp.int32(0x3F))
        keybits = m | jnp.int32(0x3F)
        vbits = jnp.where(keybits < 0, keybits ^ jnp.int32(0x7FFFFFFF), keybits)
        val_k = jax.lax.bitcast_convert_type(vbits, jnp.float32)
        top_idx = jnp.where(kiota == k, idx_k, top_idx)
        top_vals = jnp.where(kiota == k, val_k, top_vals)

    e = jnp.exp(top_vals - top_vals[0:1, :])
    weights_ref[...] = e / jnp.sum(e, axis=0, keepdims=True)
    experts_ref[...] = top_idx


def kernel(hidden_states, gate_w, gate_b):
    B, S, D = hidden_states.shape
    E = gate_w.shape[0]
    N = B * S
    x = hidden_states.reshape(N, D)
    wt = gate_w.T
    b = gate_b.reshape(1, E)

    grid = (N // _T,)
    logits, weights_t, experts_t = pl.pallas_call(
        _router_kernel,
        grid=grid,
        in_specs=[
            pl.BlockSpec((_T, D), lambda i: (i, 0)),
            pl.BlockSpec((D, E), lambda i: (0, 0)),
            pl.BlockSpec((1, E), lambda i: (0, 0)),
        ],
        out_specs=[
            pl.BlockSpec((_T, E), lambda i: (i, 0)),
            pl.BlockSpec((_K, _T), lambda i: (0, i)),
            pl.BlockSpec((_K, _T), lambda i: (0, i)),
        ],
        out_shape=[
            jax.ShapeDtypeStruct((N, E), jnp.float32),
            jax.ShapeDtypeStruct((_K, N), jnp.float32),
            jax.ShapeDtypeStruct((_K, N), jnp.int32),
        ],
    )(x, wt, b)

    return (weights_t.T.reshape(B, S, _K),
            experts_t.T.reshape(B, S, _K),
            logits.reshape(B, S, E))
